# baseline (speedup 1.0000x reference)
import functools

import jax
import jax.numpy as jnp
from jax import lax
from jax.experimental import pallas as pl
from jax.experimental.pallas import tpu as pltpu
from jax.experimental.pallas import tpu_sc as plsc

VOCAB = 100000
D_MODEL = 64
BATCH = 1024
NCTX = 10

_NC = 2
_NS = 16
_NW = _NC * _NS
_BPW = BATCH // _NW
_IPW = _BPW * NCTX
_CH = 4
_CHW = _IPW // _CH
_NVREG = D_MODEL // 16


def _sc_gather_mean(idx_flat, emb):
    mesh = plsc.VectorSubcoreMesh(core_axis_name="c", subcore_axis_name="s")

    @functools.partial(
        pl.kernel,
        out_type=jax.ShapeDtypeStruct((BATCH, D_MODEL), jnp.float32),
        mesh=mesh,
        scratch_types=[
            pltpu.VMEM((_IPW,), jnp.int32),
            pltpu.VMEM((_IPW, D_MODEL), jnp.float32),
            pltpu.VMEM((_BPW, D_MODEL), jnp.float32),
            pltpu.SemaphoreType.DMA,
        ],
        compiler_params=pltpu.CompilerParams(use_tc_tiling_on_sc=False),
    )
    def k(idx_hbm, emb_hbm, out_hbm, idx_v, rows_v, acc_v, sem):
        wid = lax.axis_index("s") * _NC + lax.axis_index("c")
        base = wid * _IPW
        pltpu.sync_copy(idx_hbm.at[pl.ds(base, _IPW)], idx_v)
        copies = [
            pltpu.async_copy(
                emb_hbm.at[idx_v.at[pl.ds(c * _CHW, _CHW)]],
                rows_v.at[pl.ds(c * _CHW, _CHW)],
                sem,
            )
            for c in range(_CH)
        ]
        for cp in copies:
            cp.wait()

        def body(b, carry):
            for v in range(_NVREG):
                s = rows_v[b * NCTX, pl.ds(v * 16, 16)]
                for j in range(1, NCTX):
                    s = s + rows_v[b * NCTX + j, pl.ds(v * 16, 16)]
                acc_v[b, pl.ds(v * 16, 16)] = s * (1.0 / NCTX)
            return carry

        lax.fori_loop(0, _BPW, body, 0)
        pltpu.sync_copy(acc_v, out_hbm.at[pl.ds(wid * _BPW, _BPW)])

    return k(idx_flat, emb)


def _tc_project(ctx_mean, W):
    VB = 2048
    grid = pl.cdiv(VOCAB, VB)

    def mm(ctx_ref, w_ref, o_ref):
        o_ref[...] = lax.dot_general(
            ctx_ref[...], w_ref[...],
            (((1,), (1,)), ((), ())),
            preferred_element_type=jnp.float32,
        )

    return pl.pallas_call(
        mm,
        grid=(grid,),
        in_specs=[
            pl.BlockSpec((BATCH, D_MODEL), lambda j: (0, 0)),
            pl.BlockSpec((VB, D_MODEL), lambda j: (j, 0)),
        ],
        out_specs=pl.BlockSpec((BATCH, VB), lambda j: (0, j)),
        out_shape=jax.ShapeDtypeStruct((BATCH, VOCAB), jnp.float32),
        compiler_params=pltpu.CompilerParams(
            dimension_semantics=("parallel",),
        ),
    )(ctx_mean, W)


def kernel(context, emb, W):
    idx_flat = context.reshape(-1).astype(jnp.int32)
    ctx_mean = _sc_gather_mean(idx_flat, emb)
    return _tc_project(ctx_mean, W)


# speedup vs baseline: 1.0055x; 1.0055x over previous
import functools

import jax
import jax.numpy as jnp
from jax import lax
from jax.experimental import pallas as pl
from jax.experimental.pallas import tpu as pltpu
from jax.experimental.pallas import tpu_sc as plsc

VOCAB = 100000
D_MODEL = 64
BATCH = 1024
NCTX = 10

_NC = 2
_NS = 16
_NW = _NC * _NS
_BPW = BATCH // _NW
_IPW = _BPW * NCTX
_CH = 4
_CHW = _IPW // _CH
_NVREG = D_MODEL // 16


def _sc_gather_mean(idx_flat, emb128):
    mesh = plsc.VectorSubcoreMesh(core_axis_name="c", subcore_axis_name="s")

    @functools.partial(
        pl.kernel,
        out_type=jax.ShapeDtypeStruct((BATCH, D_MODEL), jnp.float32),
        mesh=mesh,
        scratch_types=[
            pltpu.VMEM((_IPW,), jnp.int32),
            pltpu.VMEM((_IPW, 128), jnp.float32),
            pltpu.VMEM((_BPW, D_MODEL), jnp.float32),
            pltpu.SemaphoreType.DMA,
        ],
    )
    def k(idx_hbm, emb_hbm, out_hbm, idx_v, rows_v, acc_v, sem):
        wid = lax.axis_index("s") * _NC + lax.axis_index("c")
        base = wid * _IPW
        pltpu.sync_copy(idx_hbm.at[pl.ds(base, _IPW)], idx_v)
        copies = [
            pltpu.async_copy(
                emb_hbm.at[idx_v.at[pl.ds(c * _CHW, _CHW)]],
                rows_v.at[pl.ds(c * _CHW, _CHW)],
                sem,
            )
            for c in range(_CH)
        ]
        for cp in copies:
            cp.wait()

        def body(b, carry):
            for v in range(_NVREG):
                s = rows_v[b * NCTX, pl.ds(v * 16, 16)]
                for j in range(1, NCTX):
                    s = s + rows_v[b * NCTX + j, pl.ds(v * 16, 16)]
                acc_v[b, pl.ds(v * 16, 16)] = s * (1.0 / NCTX)
            return carry

        lax.fori_loop(0, _BPW, body, 0)
        pltpu.sync_copy(acc_v, out_hbm.at[pl.ds(wid * _BPW, _BPW)])

    return k(idx_flat, emb128)


def _tc_project(ctx_mean, W):
    VB = 2048
    grid = pl.cdiv(VOCAB, VB)

    def mm(ctx_ref, w_ref, o_ref):
        o_ref[...] = lax.dot_general(
            ctx_ref[...], w_ref[...],
            (((1,), (1,)), ((), ())),
            preferred_element_type=jnp.float32,
        )

    return pl.pallas_call(
        mm,
        grid=(grid,),
        in_specs=[
            pl.BlockSpec((BATCH, D_MODEL), lambda j: (0, 0)),
            pl.BlockSpec((VB, D_MODEL), lambda j: (j, 0)),
        ],
        out_specs=pl.BlockSpec((BATCH, VB), lambda j: (0, j)),
        out_shape=jax.ShapeDtypeStruct((BATCH, VOCAB), jnp.float32),
        compiler_params=pltpu.CompilerParams(
            dimension_semantics=("parallel",),
        ),
    )(ctx_mean, W)


def kernel(context, emb, W):
    idx_flat = context.reshape(-1).astype(jnp.int32)
    emb128 = jnp.pad(emb, ((0, 0), (0, 128 - D_MODEL)))
    ctx_mean = _sc_gather_mean(idx_flat, emb128)
    return _tc_project(ctx_mean, W)


# speedup vs baseline: 1.0057x; 1.0003x over previous
import functools

import jax
import jax.numpy as jnp
from jax import lax
from jax.experimental import pallas as pl
from jax.experimental.pallas import tpu as pltpu
from jax.experimental.pallas import tpu_sc as plsc

VOCAB = 100000
D_MODEL = 64
BATCH = 1024
NCTX = 10

_NC = 2
_NS = 16
_NW = _NC * _NS
_BPW = BATCH // _NW
_IPW = _BPW * NCTX
_CH = 4
_CHW = _IPW // _CH
_NVREG = D_MODEL // 16


def _sc_gather_mean(idx_flat, emb128):
    mesh = plsc.VectorSubcoreMesh(core_axis_name="c", subcore_axis_name="s")

    @functools.partial(
        pl.kernel,
        out_type=jax.ShapeDtypeStruct((BATCH, D_MODEL), jnp.float32),
        mesh=mesh,
        scratch_types=[
            pltpu.VMEM((_IPW,), jnp.int32),
            pltpu.VMEM((_IPW, 128), jnp.float32),
            pltpu.VMEM((_BPW, D_MODEL), jnp.float32),
            pltpu.SemaphoreType.DMA,
        ],
        compiler_params=pltpu.CompilerParams(use_tc_tiling_on_sc=True),
    )
    def k(idx_hbm, emb_hbm, out_hbm, idx_v, rows_v, acc_v, sem):
        wid = lax.axis_index("s") * _NC + lax.axis_index("c")
        base = wid * _IPW
        pltpu.sync_copy(idx_hbm.at[pl.ds(base, _IPW)], idx_v)
        copies = [
            pltpu.async_copy(
                emb_hbm.at[idx_v.at[pl.ds(c * _CHW, _CHW)]],
                rows_v.at[pl.ds(c * _CHW, _CHW)],
                sem,
            )
            for c in range(_CH)
        ]
        for cp in copies:
            cp.wait()

        def body(b, carry):
            for v in range(_NVREG):
                s = rows_v[b * NCTX, pl.ds(v * 16, 16)]
                for j in range(1, NCTX):
                    s = s + rows_v[b * NCTX + j, pl.ds(v * 16, 16)]
                acc_v[b, pl.ds(v * 16, 16)] = s * (1.0 / NCTX)
            return carry

        lax.fori_loop(0, _BPW, body, 0)
        pltpu.sync_copy(acc_v, out_hbm.at[pl.ds(wid * _BPW, _BPW)])

    return k(idx_flat, emb128)


def _tc_project(ctx_mean, W):
    VB = 2048
    grid = pl.cdiv(VOCAB, VB)

    def mm(ctx_ref, w_ref, o_ref):
        o_ref[...] = lax.dot_general(
            ctx_ref[...], w_ref[...],
            (((1,), (1,)), ((), ())),
            preferred_element_type=jnp.float32,
        )

    return pl.pallas_call(
        mm,
        grid=(grid,),
        in_specs=[
            pl.BlockSpec((BATCH, D_MODEL), lambda j: (0, 0)),
            pl.BlockSpec((VB, D_MODEL), lambda j: (j, 0)),
        ],
        out_specs=pl.BlockSpec((BATCH, VB), lambda j: (0, j)),
        out_shape=jax.ShapeDtypeStruct((BATCH, VOCAB), jnp.float32),
        compiler_params=pltpu.CompilerParams(
            dimension_semantics=("parallel",),
        ),
    )(ctx_mean, W)


def kernel(context, emb, W):
    idx_flat = context.reshape(-1).astype(jnp.int32)
    emb128 = jnp.pad(emb, ((0, 0), (0, 128 - D_MODEL)))
    ctx_mean = _sc_gather_mean(idx_flat, emb128)
    return _tc_project(ctx_mean, W)


# speedup vs baseline: 2.8827x; 2.8662x over previous
import functools

import jax
import jax.numpy as jnp
from jax import lax
from jax.experimental import pallas as pl
from jax.experimental.pallas import tpu as pltpu
from jax.experimental.pallas import tpu_sc as plsc

VOCAB = 100000
D_MODEL = 64
BATCH = 1024
NCTX = 10

_NC = 2
_NS = 16
_NW = _NC * _NS
_BPW = BATCH // _NW
_IPW = _BPW * NCTX
_CH = 4
_CHW = _IPW // _CH
_NVREG = D_MODEL // 16


def _tc_repack(embT):
    RB = 2048
    grid = pl.cdiv(VOCAB, RB)

    def rp(in_ref, o_ref):
        t = in_ref[...].T
        o_ref[...] = jnp.concatenate([t, t], axis=1)

    return pl.pallas_call(
        rp,
        grid=(grid,),
        in_specs=[pl.BlockSpec((D_MODEL, RB), lambda i: (0, i))],
        out_specs=pl.BlockSpec((RB, 128), lambda i: (i, 0)),
        out_shape=jax.ShapeDtypeStruct((VOCAB, 128), jnp.float32),
        compiler_params=pltpu.CompilerParams(
            dimension_semantics=("parallel",),
        ),
    )(embT)


def _sc_gather_mean(idx_flat, emb128):
    mesh = plsc.VectorSubcoreMesh(core_axis_name="c", subcore_axis_name="s")

    @functools.partial(
        pl.kernel,
        out_type=jax.ShapeDtypeStruct((BATCH, D_MODEL), jnp.float32),
        mesh=mesh,
        scratch_types=[
            pltpu.VMEM((_IPW,), jnp.int32),
            pltpu.VMEM((_IPW, 128), jnp.float32),
            pltpu.VMEM((_BPW, D_MODEL), jnp.float32),
            pltpu.SemaphoreType.DMA,
        ],
        compiler_params=pltpu.CompilerParams(use_tc_tiling_on_sc=True),
    )
    def k(idx_hbm, emb_hbm, out_hbm, idx_v, rows_v, acc_v, sem):
        wid = lax.axis_index("s") * _NC + lax.axis_index("c")
        base = wid * _IPW
        pltpu.sync_copy(idx_hbm.at[pl.ds(base, _IPW)], idx_v)
        copies = [
            pltpu.async_copy(
                emb_hbm.at[idx_v.at[pl.ds(c * _CHW, _CHW)]],
                rows_v.at[pl.ds(c * _CHW, _CHW)],
                sem,
            )
            for c in range(_CH)
        ]
        for cp in copies:
            cp.wait()

        def body(b, carry):
            for v in range(_NVREG):
                s = rows_v[b * NCTX, pl.ds(v * 16, 16)]
                for j in range(1, NCTX):
                    s = s + rows_v[b * NCTX + j, pl.ds(v * 16, 16)]
                acc_v[b, pl.ds(v * 16, 16)] = s * (1.0 / NCTX)
            return carry

        lax.fori_loop(0, _BPW, body, 0)
        pltpu.sync_copy(acc_v, out_hbm.at[pl.ds(wid * _BPW, _BPW)])

    return k(idx_flat, emb128)


def _tc_project_t(ctx_mean, Wt):
    VB = 2048
    grid = pl.cdiv(VOCAB, VB)

    def mm(w_ref, ctx_ref, o_ref):
        o_ref[...] = lax.dot_general(
            w_ref[...], ctx_ref[...],
            (((0,), (1,)), ((), ())),
            preferred_element_type=jnp.float32,
        )

    return pl.pallas_call(
        mm,
        grid=(grid,),
        in_specs=[
            pl.BlockSpec((D_MODEL, VB), lambda j: (0, j)),
            pl.BlockSpec((BATCH, D_MODEL), lambda j: (0, 0)),
        ],
        out_specs=pl.BlockSpec((VB, BATCH), lambda j: (j, 0)),
        out_shape=jax.ShapeDtypeStruct((VOCAB, BATCH), jnp.float32),
        compiler_params=pltpu.CompilerParams(
            dimension_semantics=("parallel",),
        ),
    )(Wt, ctx_mean)


def kernel(context, emb, W):
    idx_flat = context.reshape(-1).astype(jnp.int32)
    embT = jnp.transpose(emb)
    Wt = jnp.transpose(W)
    emb128 = _tc_repack(embT)
    ctx_mean = _sc_gather_mean(idx_flat, emb128)
    logits_t = _tc_project_t(ctx_mean, Wt)
    return jnp.transpose(logits_t)


# speedup vs baseline: 3.1278x; 1.0850x over previous
import functools

import jax
import jax.numpy as jnp
from jax import lax
from jax.experimental import pallas as pl
from jax.experimental.pallas import tpu as pltpu
from jax.experimental.pallas import tpu_sc as plsc

VOCAB = 100000
D_MODEL = 64
BATCH = 1024
NCTX = 10

_NC = 2
_NS = 16
_NW = _NC * _NS
_BPW = BATCH // _NW
_IPW = _BPW * NCTX
_CH = 4
_CHW = _IPW // _CH
_NVREG = D_MODEL // 16


def _tc_repack(embT):
    RB = 4096
    grid = pl.cdiv(VOCAB, RB)

    def rp(in_ref, o_ref):
        r = lax.broadcasted_iota(jnp.int32, (D_MODEL, 128), 0)
        c = lax.broadcasted_iota(jnp.int32, (D_MODEL, 128), 1)
        eye2 = jnp.where((c % D_MODEL) == r, 1.0, 0.0).astype(jnp.float32)
        o_ref[...] = lax.dot_general(
            in_ref[...], eye2,
            (((0,), (0,)), ((), ())),
            preferred_element_type=jnp.float32,
        )

    return pl.pallas_call(
        rp,
        grid=(grid,),
        in_specs=[pl.BlockSpec((D_MODEL, RB), lambda i: (0, i))],
        out_specs=pl.BlockSpec((RB, 128), lambda i: (i, 0)),
        name="repack",
        out_shape=jax.ShapeDtypeStruct((VOCAB, 128), jnp.float32),
        compiler_params=pltpu.CompilerParams(
            dimension_semantics=("parallel",),
        ),
    )(embT)


def _sc_gather_mean(idx_flat, emb128):
    mesh = plsc.VectorSubcoreMesh(core_axis_name="c", subcore_axis_name="s")

    @functools.partial(
        pl.kernel,
        out_type=jax.ShapeDtypeStruct((BATCH, D_MODEL), jnp.float32),
        mesh=mesh,
        scratch_types=[
            pltpu.VMEM((_IPW,), jnp.int32),
            pltpu.VMEM((_IPW, 128), jnp.float32),
            pltpu.VMEM((_BPW, D_MODEL), jnp.float32),
            pltpu.SemaphoreType.DMA,
        ],
        compiler_params=pltpu.CompilerParams(use_tc_tiling_on_sc=True),
    )
    def k(idx_hbm, emb_hbm, out_hbm, idx_v, rows_v, acc_v, sem):
        wid = lax.axis_index("s") * _NC + lax.axis_index("c")
        base = wid * _IPW
        pltpu.sync_copy(idx_hbm.at[pl.ds(base, _IPW)], idx_v)
        copies = [
            pltpu.async_copy(
                emb_hbm.at[idx_v.at[pl.ds(c * _CHW, _CHW)]],
                rows_v.at[pl.ds(c * _CHW, _CHW)],
                sem,
            )
            for c in range(_CH)
        ]
        for cp in copies:
            cp.wait()

        def body(b, carry):
            for v in range(_NVREG):
                s = rows_v[b * NCTX, pl.ds(v * 16, 16)]
                for j in range(1, NCTX):
                    s = s + rows_v[b * NCTX + j, pl.ds(v * 16, 16)]
                acc_v[b, pl.ds(v * 16, 16)] = s * (1.0 / NCTX)
            return carry

        lax.fori_loop(0, _BPW, body, 0)
        pltpu.sync_copy(acc_v, out_hbm.at[pl.ds(wid * _BPW, _BPW)])

    return k(idx_flat, emb128)


def _tc_project_t(ctx_mean, Wt):
    VB = 2048
    grid = pl.cdiv(VOCAB, VB)

    def mm(w_ref, ctx_ref, o_ref):
        o_ref[...] = lax.dot_general(
            w_ref[...], ctx_ref[...],
            (((0,), (1,)), ((), ())),
            preferred_element_type=jnp.float32,
        )

    return pl.pallas_call(
        mm,
        grid=(grid,),
        in_specs=[
            pl.BlockSpec((D_MODEL, VB), lambda j: (0, j)),
            pl.BlockSpec((BATCH, D_MODEL), lambda j: (0, 0)),
        ],
        out_specs=pl.BlockSpec((VB, BATCH), lambda j: (j, 0)),
        out_shape=jax.ShapeDtypeStruct((VOCAB, BATCH), jnp.float32),
        compiler_params=pltpu.CompilerParams(
            dimension_semantics=("parallel",),
        ),
    )(Wt, ctx_mean)


def kernel(context, emb, W):
    idx_flat = context.reshape(-1).astype(jnp.int32)
    embT = jnp.transpose(emb)
    Wt = jnp.transpose(W)
    emb128 = _tc_repack(embT)
    ctx_mean = _sc_gather_mean(idx_flat, emb128)
    logits_t = _tc_project_t(ctx_mean, Wt)
    return jnp.transpose(logits_t)


# speedup vs baseline: 3.1452x; 1.0056x over previous
import functools

import jax
import jax.numpy as jnp
from jax import lax
from jax.experimental import pallas as pl
from jax.experimental.pallas import tpu as pltpu
from jax.experimental.pallas import tpu_sc as plsc

VOCAB = 100000
D_MODEL = 64
BATCH = 1024
NCTX = 10

_NC = 2
_NS = 16
_NW = _NC * _NS
_BPW = BATCH // _NW
_IPW = _BPW * NCTX
_CH = 4
_CHW = _IPW // _CH
_NVREG = D_MODEL // 16


def _tc_repack(embT):
    RB = 4096
    grid = pl.cdiv(VOCAB, RB)

    def rp(in_ref, o_ref):
        r = lax.broadcasted_iota(jnp.int32, (D_MODEL, 128), 0)
        c = lax.broadcasted_iota(jnp.int32, (D_MODEL, 128), 1)
        eye2 = jnp.where((c % D_MODEL) == r, 1.0, 0.0).astype(jnp.float32)
        o_ref[...] = lax.dot_general(
            in_ref[...], eye2,
            (((0,), (0,)), ((), ())),
            preferred_element_type=jnp.float32,
        )

    return pl.pallas_call(
        rp,
        grid=(grid,),
        in_specs=[pl.BlockSpec((D_MODEL, RB), lambda i: (0, i))],
        out_specs=pl.BlockSpec((RB, 128), lambda i: (i, 0)),
        name="repack",
        out_shape=jax.ShapeDtypeStruct((VOCAB, 128), jnp.float32),
        compiler_params=pltpu.CompilerParams(
            dimension_semantics=("parallel",),
        ),
    )(embT)


def _sc_gather_mean(idx_flat, emb128):
    mesh = plsc.VectorSubcoreMesh(core_axis_name="c", subcore_axis_name="s")

    @functools.partial(
        pl.kernel,
        out_type=jax.ShapeDtypeStruct((BATCH, D_MODEL), jnp.float32),
        mesh=mesh,
        scratch_types=[
            pltpu.VMEM((_IPW,), jnp.int32),
            pltpu.VMEM((_IPW, 128), jnp.float32),
            pltpu.VMEM((_BPW, D_MODEL), jnp.float32),
            pltpu.SemaphoreType.DMA,
        ],
        compiler_params=pltpu.CompilerParams(use_tc_tiling_on_sc=True),
    )
    def k(idx_hbm, emb_hbm, out_hbm, idx_v, rows_v, acc_v, sem):
        wid = lax.axis_index("s") * _NC + lax.axis_index("c")
        base = wid * _IPW
        pltpu.sync_copy(idx_hbm.at[pl.ds(base, _IPW)], idx_v)
        copies = [
            pltpu.async_copy(
                emb_hbm.at[idx_v.at[pl.ds(c * _CHW, _CHW)]],
                rows_v.at[pl.ds(c * _CHW, _CHW)],
                sem,
            )
            for c in range(_CH)
        ]
        for cp in copies:
            cp.wait()

        def body(b, carry):
            for v in range(_NVREG):
                s = rows_v[b * NCTX, pl.ds(v * 16, 16)]
                for j in range(1, NCTX):
                    s = s + rows_v[b * NCTX + j, pl.ds(v * 16, 16)]
                acc_v[b, pl.ds(v * 16, 16)] = s * (1.0 / NCTX)
            return carry

        lax.fori_loop(0, _BPW, body, 0)
        pltpu.sync_copy(acc_v, out_hbm.at[pl.ds(wid * _BPW, _BPW)])

    return k(idx_flat, emb128)


def _tc_project_t(ctx_mean, Wt):
    VB = 4096
    grid = pl.cdiv(VOCAB, VB)

    def mm(w_ref, ctx_ref, o_ref):
        o_ref[...] = lax.dot_general(
            w_ref[...], ctx_ref[...],
            (((0,), (1,)), ((), ())),
            preferred_element_type=jnp.float32,
        )

    return pl.pallas_call(
        mm,
        grid=(grid,),
        in_specs=[
            pl.BlockSpec((D_MODEL, VB), lambda j: (0, j)),
            pl.BlockSpec((BATCH, D_MODEL), lambda j: (0, 0)),
        ],
        out_specs=pl.BlockSpec((VB, BATCH), lambda j: (j, 0)),
        out_shape=jax.ShapeDtypeStruct((VOCAB, BATCH), jnp.float32),
        compiler_params=pltpu.CompilerParams(
            dimension_semantics=("parallel",),
        ),
    )(Wt, ctx_mean)


def kernel(context, emb, W):
    idx_flat = context.reshape(-1).astype(jnp.int32)
    embT = jnp.transpose(emb)
    Wt = jnp.transpose(W)
    emb128 = _tc_repack(embT)
    ctx_mean = _sc_gather_mean(idx_flat, emb128)
    logits_t = _tc_project_t(ctx_mean, Wt)
    return jnp.transpose(logits_t)


# speedup vs baseline: 3.1757x; 1.0097x over previous
import functools

import jax
import jax.numpy as jnp
from jax import lax
from jax.experimental import pallas as pl
from jax.experimental.pallas import tpu as pltpu
from jax.experimental.pallas import tpu_sc as plsc

VOCAB = 100000
D_MODEL = 64
BATCH = 1024
NCTX = 10

_NC = 2
_NS = 16
_NW = _NC * _NS
_BPW = BATCH // _NW
_IPW = _BPW * NCTX
_CH = 4
_CHW = _IPW // _CH
_NVREG = D_MODEL // 16


def _tc_repack(embT):
    RB = 8192
    grid = pl.cdiv(VOCAB, RB)

    def rp(in_ref, o_ref):
        t = in_ref[...].T
        o_ref[...] = jnp.concatenate([t, t], axis=1)

    return pl.pallas_call(
        rp,
        grid=(grid,),
        in_specs=[pl.BlockSpec((D_MODEL, RB), lambda i: (0, i))],
        out_specs=pl.BlockSpec((RB, 128), lambda i: (i, 0)),
        name="repack",
        out_shape=jax.ShapeDtypeStruct((VOCAB, 128), jnp.float32),
        compiler_params=pltpu.CompilerParams(
            dimension_semantics=("parallel",),
        ),
    )(embT)


def _sc_gather_mean(idx_flat, emb128):
    mesh = plsc.VectorSubcoreMesh(core_axis_name="c", subcore_axis_name="s")

    @functools.partial(
        pl.kernel,
        out_type=jax.ShapeDtypeStruct((BATCH, D_MODEL), jnp.float32),
        mesh=mesh,
        scratch_types=[
            pltpu.VMEM((_IPW,), jnp.int32),
            pltpu.VMEM((_IPW, 128), jnp.float32),
            pltpu.VMEM((_BPW, D_MODEL), jnp.float32),
            pltpu.SemaphoreType.DMA,
        ],
        compiler_params=pltpu.CompilerParams(use_tc_tiling_on_sc=True),
    )
    def k(idx_hbm, emb_hbm, out_hbm, idx_v, rows_v, acc_v, sem):
        wid = lax.axis_index("s") * _NC + lax.axis_index("c")
        base = wid * _IPW
        pltpu.sync_copy(idx_hbm.at[pl.ds(base, _IPW)], idx_v)
        copies = [
            pltpu.async_copy(
                emb_hbm.at[idx_v.at[pl.ds(c * _CHW, _CHW)]],
                rows_v.at[pl.ds(c * _CHW, _CHW)],
                sem,
            )
            for c in range(_CH)
        ]
        for cp in copies:
            cp.wait()

        def body(b, carry):
            for v in range(_NVREG):
                s = rows_v[b * NCTX, pl.ds(v * 16, 16)]
                for j in range(1, NCTX):
                    s = s + rows_v[b * NCTX + j, pl.ds(v * 16, 16)]
                acc_v[b, pl.ds(v * 16, 16)] = s * (1.0 / NCTX)
            return carry

        lax.fori_loop(0, _BPW, body, 0)
        pltpu.sync_copy(acc_v, out_hbm.at[pl.ds(wid * _BPW, _BPW)])

    return k(idx_flat, emb128)


def _tc_project_t(ctx_mean, Wt):
    VB = 5120
    grid = pl.cdiv(VOCAB, VB)

    def mm(w_ref, ctx_ref, o_ref):
        o_ref[...] = lax.dot_general(
            w_ref[...], ctx_ref[...],
            (((0,), (1,)), ((), ())),
            preferred_element_type=jnp.float32,
        )

    return pl.pallas_call(
        mm,
        grid=(grid,),
        in_specs=[
            pl.BlockSpec((D_MODEL, VB), lambda j: (0, j)),
            pl.BlockSpec((BATCH, D_MODEL), lambda j: (0, 0)),
        ],
        out_specs=pl.BlockSpec((VB, BATCH), lambda j: (j, 0)),
        out_shape=jax.ShapeDtypeStruct((VOCAB, BATCH), jnp.float32),
        compiler_params=pltpu.CompilerParams(
            dimension_semantics=("parallel",),
        ),
    )(Wt, ctx_mean)


def kernel(context, emb, W):
    idx_flat = context.reshape(-1).astype(jnp.int32)
    embT = jnp.transpose(emb)
    Wt = jnp.transpose(W)
    emb128 = _tc_repack(embT)
    ctx_mean = _sc_gather_mean(idx_flat, emb128)
    logits_t = _tc_project_t(ctx_mean, Wt)
    return jnp.transpose(logits_t)
